# baseline (speedup 1.0000x reference)
import functools
import math

import jax
import jax.numpy as jnp
from jax import lax
from jax.experimental import pallas as pl
from jax.experimental.pallas import tpu as pltpu
from jax.experimental.pallas import tpu_sc as plsc

_LANES = 16
_NUM_WORKERS = 32


def kernel(input_ids, token_table, pos_table):
    B, S = input_ids.shape
    V, D = token_table.shape
    N = B * S
    scale = math.sqrt(float(D))

    n_per_w = N // _NUM_WORKERS
    chunk = 64
    n_chunks = n_per_w // chunk

    ids_flat = input_ids.reshape(N)

    mesh = plsc.VectorSubcoreMesh(core_axis_name="c", subcore_axis_name="s")

    @functools.partial(
        pl.kernel,
        mesh=mesh,
        out_type=jax.ShapeDtypeStruct((N, D), jnp.float32),
        scratch_types=[
            pltpu.VMEM((n_per_w,), jnp.int32),
            pltpu.VMEM((chunk, D), jnp.float32),
            pltpu.VMEM((chunk, D), jnp.float32),
            pltpu.SemaphoreType.DMA,
        ],
    )
    def body(ids_hbm, tok_hbm, pos_hbm, out_hbm, idx_v, tok_v, pos_v, sem):
        wid = lax.axis_index("s") * 2 + lax.axis_index("c")
        base = wid * n_per_w
        s0 = lax.rem(base, S)
        pltpu.sync_copy(ids_hbm.at[pl.ds(base, n_per_w)], idx_v)
        for c in range(n_chunks):
            off = c * chunk
            gather = pltpu.async_copy(
                tok_hbm.at[idx_v.at[pl.ds(off, chunk)]], tok_v, sem)
            pltpu.sync_copy(pos_hbm.at[pl.ds(s0 + off, chunk)], pos_v)
            gather.wait()

            def row(r, _):
                def col(k, _):
                    sl = pl.ds(k * _LANES, _LANES)
                    tok_v[r, sl] = tok_v[r, sl] * scale + pos_v[r, sl]
                    return 0
                return lax.fori_loop(0, D // _LANES, col, 0)

            lax.fori_loop(0, chunk, row, 0)
            pltpu.sync_copy(tok_v, out_hbm.at[pl.ds(base + off, chunk)])

    out = body(ids_flat, token_table, pos_table)
    return out.reshape(B, S, D)


# speedup vs baseline: 2.1412x; 2.1412x over previous
import functools
import math

import jax
import jax.numpy as jnp
from jax import lax
from jax.experimental import pallas as pl
from jax.experimental.pallas import tpu as pltpu
from jax.experimental.pallas import tpu_sc as plsc

_LANES = 16
_NUM_WORKERS = 32


def kernel(input_ids, token_table, pos_table):
    B, S = input_ids.shape
    V, D = token_table.shape
    N = B * S
    scale = math.sqrt(float(D))
    s_per_w = S // _NUM_WORKERS
    nvec = D // _LANES

    mesh = plsc.VectorSubcoreMesh(core_axis_name="c", subcore_axis_name="s")

    @functools.partial(
        pl.kernel,
        mesh=mesh,
        out_type=jax.ShapeDtypeStruct((N, D), jnp.float32),
        scratch_types=[
            pltpu.VMEM((B, s_per_w), jnp.int32),
            pltpu.VMEM((s_per_w, D), jnp.float32),
            pltpu.VMEM((s_per_w, D), jnp.float32),
            pltpu.VMEM((s_per_w, D), jnp.float32),
            pltpu.SemaphoreType.DMA,
            pltpu.SemaphoreType.DMA,
            pltpu.SemaphoreType.DMA,
            pltpu.SemaphoreType.DMA,
        ],
    )
    def body(ids_hbm, tok_hbm, pos_hbm, out_hbm, idx_v, pos_v, t0, t1,
             g0, g1, o0, o1):
        wid = lax.axis_index("s") * 2 + lax.axis_index("c")
        s0 = wid * s_per_w
        for b in range(B):
            pltpu.sync_copy(ids_hbm.at[pl.ds(b * S + s0, s_per_w)],
                            idx_v.at[b])
        pltpu.sync_copy(pos_hbm.at[pl.ds(s0, s_per_w)], pos_v)

        tbufs = [t0, t1]
        gsems = [g0, g1]
        osems = [o0, o1]
        gathers = [None, None]
        stores = [None, None]
        gathers[0] = pltpu.async_copy(tok_hbm.at[idx_v.at[0]], t0, g0)
        for b in range(B):
            cur = b % 2
            nxt = (b + 1) % 2
            if b + 1 < B:
                if stores[nxt] is not None:
                    stores[nxt].wait()
                gathers[nxt] = pltpu.async_copy(
                    tok_hbm.at[idx_v.at[b + 1]], tbufs[nxt], gsems[nxt])
            gathers[cur].wait()
            buf = tbufs[cur]

            def row(r, _, buf=buf):
                for k in range(nvec):
                    sl = pl.ds(k * _LANES, _LANES)
                    buf[r, sl] = buf[r, sl] * scale + pos_v[r, sl]
                return 0

            lax.fori_loop(0, s_per_w, row, 0)
            stores[cur] = pltpu.async_copy(
                buf, out_hbm.at[pl.ds(b * S + s0, s_per_w)], osems[cur])
        stores[0].wait()
        stores[1].wait()

    out = body(input_ids.reshape(N), token_table, pos_table)
    return out.reshape(B, S, D)
